# baseline (speedup 1.0000x reference)
import functools

import jax
import jax.numpy as jnp
from jax import lax
from jax.experimental import pallas as pl
from jax.experimental.pallas import tpu as pltpu
from jax.experimental.pallas import tpu_sc as plsc

N_TOK = 16384
D = 4096
E = 64
K = 8
M_BLK = 1024
NCHUNK = 4
TOK_C = N_TOK // NCHUNK

_info = plsc.get_sparse_core_info()
_NC, _NS, _L = _info.num_cores, _info.num_subcores, _info.num_lanes
_NW = _NC * _NS
_ROWS_W = TOK_C // _NW
_mesh = plsc.VectorSubcoreMesh(core_axis_name="c", subcore_axis_name="s")


def _mm_block(x_ref, wt_ref, b_ref, rw_ref):
    rw_ref[...] = (
        jnp.dot(x_ref[...], wt_ref[...], preferred_element_type=jnp.float32)
        + b_ref[...]
    )


def _tc_matmul(x, wt, b2, ci):
    grid = (TOK_C // M_BLK,)
    off = ci * (TOK_C // M_BLK)
    return pl.pallas_call(
        _mm_block,
        grid=grid,
        in_specs=[
            pl.BlockSpec((M_BLK, D), lambda i: (off + i, 0)),
            pl.BlockSpec((D, E), lambda i: (0, 0)),
            pl.BlockSpec((1, E), lambda i: (0, 0)),
        ],
        out_specs=pl.BlockSpec((M_BLK, E), lambda i: (i, 0)),
        out_shape=jax.ShapeDtypeStruct((TOK_C, E), jnp.float32),
        compiler_params=pltpu.CompilerParams(
            dimension_semantics=("arbitrary",),
        ),
    )(x, wt, b2)


@functools.partial(
    pl.kernel,
    mesh=_mesh,
    out_type=jax.ShapeDtypeStruct((TOK_C * E,), jnp.float32),
    scratch_types=[
        pltpu.VMEM((_ROWS_W * E,), jnp.float32),
        pltpu.VMEM((_ROWS_W * E,), jnp.float32),
    ],
    compiler_params=pltpu.CompilerParams(needs_layout_passes=False),
)
def _sc_gates(rw_hbm, gates_hbm, rw_v, gates_v):
    wid = lax.axis_index("s") * _NC + lax.axis_index("c")
    base = wid * _ROWS_W
    pltpu.sync_copy(rw_hbm.at[pl.ds(base * E, _ROWS_W * E)], rw_v)

    lane = lax.iota(jnp.int32, _L)
    hi_mask = lane >= (_L - K)

    def row(r, carry):
        off = r * E
        v = [rw_v[pl.ds(off + i * _L, _L)] for i in range(E // _L)]
        sv = [jnp.sort(u) for u in v]
        a = jnp.sort(jnp.maximum(sv[0], jnp.flip(sv[1])))
        b2 = jnp.sort(jnp.maximum(sv[2], jnp.flip(sv[3])))
        c = jnp.sort(jnp.maximum(a, jnp.flip(b2)))
        m0 = c[_L - 1]
        t8 = c[_L - K]
        es = jnp.where(hi_mask, jnp.exp(c - m0), 0.0)
        s_vec = jnp.full((_L,), jnp.sum(es), jnp.float32)
        rinv = jnp.full((_L,), 1.0, jnp.float32) / s_vec
        for i in range(E // _L):
            gv = jnp.where(v[i] >= t8, jnp.exp(v[i] - m0) * rinv, 0.0)
            gates_v[pl.ds(off + i * _L, _L)] = gv
        return carry

    lax.fori_loop(0, _ROWS_W, row, 0)
    pltpu.sync_copy(gates_v, gates_hbm.at[pl.ds(base * E, _ROWS_W * E)])


@jax.jit
def kernel(x, W, b):
    wt = W.T
    b2 = b.reshape(1, E)
    rws = []
    gs = []
    for ci in range(NCHUNK):
        rw_c = _tc_matmul(x, wt, b2, ci)
        rws.append(rw_c)
        gs.append(_sc_gates(rw_c.reshape(TOK_C * E)).reshape(TOK_C, E))
    rw = jnp.concatenate(rws, axis=0)
    gates = jnp.concatenate(gs, axis=0)
    return (gates, rw)


# speedup vs baseline: 1.6922x; 1.6922x over previous
import functools

import jax
import jax.numpy as jnp
from jax import lax
from jax.experimental import pallas as pl
from jax.experimental.pallas import tpu as pltpu

N_TOK = 16384
D = 4096
E = 64
K = 8
M_BLK = 1024


def _router_block(x_ref, wt_ref, b_ref, rw_ref, gates_ref):
    acc = jnp.dot(x_ref[...], wt_ref[...], preferred_element_type=jnp.float32)
    rw = acc + b_ref[...]
    rw_ref[...] = rw

    cur = rw
    t = jnp.max(cur, axis=1, keepdims=True)
    m0 = t
    s = jnp.ones_like(t)
    for _ in range(K - 1):
        cur = jnp.where(cur == t, -jnp.inf, cur)
        t = jnp.max(cur, axis=1, keepdims=True)
        s = s + jnp.exp(t - m0)
    rinv = 1.0 / s
    gates_ref[...] = jnp.where(rw >= t, jnp.exp(rw - m0) * rinv, 0.0)


@jax.jit
def kernel(x, W, b):
    wt = W.T
    b2 = b.reshape(1, E)
    grid = (N_TOK // M_BLK,)
    rw, gates = pl.pallas_call(
        _router_block,
        grid=grid,
        in_specs=[
            pl.BlockSpec((M_BLK, D), lambda i: (i, 0)),
            pl.BlockSpec((D, E), lambda i: (0, 0)),
            pl.BlockSpec((1, E), lambda i: (0, 0)),
        ],
        out_specs=[
            pl.BlockSpec((M_BLK, E), lambda i: (i, 0)),
            pl.BlockSpec((M_BLK, E), lambda i: (i, 0)),
        ],
        out_shape=[
            jax.ShapeDtypeStruct((N_TOK, E), jnp.float32),
            jax.ShapeDtypeStruct((N_TOK, E), jnp.float32),
        ],
        compiler_params=pltpu.CompilerParams(
            dimension_semantics=("arbitrary",),
        ),
    )(x, wt, b2)
    return (gates, rw)


# speedup vs baseline: 1.6941x; 1.0011x over previous
import jax
import jax.numpy as jnp
from jax.experimental import pallas as pl
from jax.experimental.pallas import tpu as pltpu

N_TOK = 16384
D = 4096
E = 64
K = 8
M_BLK = 1024


def _router_block(x_ref, wt_ref, b_ref, rw_ref, gates_ref):
    acc = jnp.dot(x_ref[...], wt_ref[...], preferred_element_type=jnp.float32)
    rw = acc + b_ref[...]
    rw_ref[...] = rw

    cur = rw
    t = jnp.max(cur, axis=1, keepdims=True)
    m0 = t
    s = jnp.ones_like(t)
    for _ in range(K - 1):
        cur = jnp.where(cur == t, -jnp.inf, cur)
        t = jnp.max(cur, axis=1, keepdims=True)
        s = s + jnp.exp(t - m0)
    rinv = 1.0 / s
    gates_ref[...] = jnp.where(rw >= t, jnp.exp(rw - m0) * rinv, 0.0)


@jax.jit
def kernel(x, W, b):
    wt = W.T
    b2 = b.reshape(1, E)
    grid = (N_TOK // M_BLK,)
    rw, gates = pl.pallas_call(
        _router_block,
        grid=grid,
        in_specs=[
            pl.BlockSpec((M_BLK, D), lambda i: (i, 0)),
            pl.BlockSpec((D, E), lambda i: (0, 0)),
            pl.BlockSpec((1, E), lambda i: (0, 0)),
        ],
        out_specs=[
            pl.BlockSpec((M_BLK, E), lambda i: (i, 0)),
            pl.BlockSpec((M_BLK, E), lambda i: (i, 0)),
        ],
        out_shape=[
            jax.ShapeDtypeStruct((N_TOK, E), jnp.float32),
            jax.ShapeDtypeStruct((N_TOK, E), jnp.float32),
        ],
        compiler_params=pltpu.CompilerParams(
            dimension_semantics=("arbitrary",),
        ),
    )(x, wt, b2)
    return (gates, rw)
